# Initial kernel scaffold; baseline (speedup 1.0000x reference)
import jax
import jax.numpy as jnp
from jax.experimental import pallas as pl


def kernel(t, x_user, x_item, edge_index_u2i, edge_index_i2u, W1_u2i, b1_u2i, W1_i2u, b1_i2u, W2_u2i, b2_u2i, W2_i2u, b2_i2u, P_user, pb_user, P_item, pb_item):
    raise NotImplementedError("write your pallas kernel here")


# speedup vs baseline: 12.9777x; 12.9777x over previous
import functools

import jax
import jax.numpy as jnp
from jax import lax
from jax.experimental import pallas as pl
from jax.experimental.pallas import tpu as pltpu
from jax.experimental.pallas import tpu_sc as plsc

N = 25000
NPAD = 25088
E = 400000
D = 128
H = 64
OUT = 2

NW = 32
EW = E // NW
C = 128
NCH = 98
EPAD = NCH * C
RPT = NPAD // 16

_mesh = plsc.VectorSubcoreMesh(core_axis_name="c", subcore_axis_name="s")
_sc_params = pltpu.CompilerParams(use_tc_tiling_on_sc=False)


@functools.partial(
    pl.kernel,
    out_type=jax.ShapeDtypeStruct((2, 4, NPAD), jnp.float32),
    mesh=_mesh,
    scratch_types=[
        pltpu.VMEM((NCH, C), jnp.int32),
        pltpu.VMEM((128,), jnp.float32),
        pltpu.VMEM_SHARED((NPAD,), jnp.float32),
        pltpu.VMEM_SHARED((NPAD,), jnp.float32),
        pltpu.VMEM_SHARED((NPAD,), jnp.float32),
        pltpu.VMEM_SHARED((NPAD,), jnp.float32),
    ],
    compiler_params=_sc_params,
)
def _hist(i0, i1, i2, i3, zeros1, out, idx, ones, a0, a1, a2, a3):
    cid = lax.axis_index("c")
    sid = lax.axis_index("s")
    wid = cid * 16 + sid
    r0 = sid * RPT
    accs = (a0, a1, a2, a3)
    for acc in accs:
        pltpu.sync_copy(zeros1, acc.at[pl.ds(r0, RPT)])
    for i in range(8):
        ones[pl.ds(i * 16, 16)] = jnp.full((16,), 1.0, jnp.float32)
    plsc.subcore_barrier()
    for src, acc in zip((i0, i1, i2, i3), accs):
        pltpu.sync_copy(src.at[wid], idx)

        def body(c, _, acc=acc):
            pltpu.sync_copy(ones.at[pl.ds(0, C)], acc.at[idx.at[c]], add=True)
            return 0

        lax.fori_loop(0, NCH, body, 0)
    plsc.subcore_barrier()
    for a, acc in enumerate(accs):
        pltpu.sync_copy(acc.at[pl.ds(r0, RPT)], out.at[cid, a, pl.ds(r0, RPT)])


@functools.partial(
    pl.kernel,
    out_type=jax.ShapeDtypeStruct((2, NPAD, H), jnp.float32),
    mesh=_mesh,
    scratch_types=[
        pltpu.VMEM((2, C), jnp.int32),
        pltpu.VMEM((2, C), jnp.int32),
        pltpu.VMEM((C, H), jnp.float32),
        pltpu.VMEM_SHARED((NPAD, H), jnp.float32),
        pltpu.SemaphoreType.DMA,
    ],
    compiler_params=_sc_params,
)
def _scatter(y, srcs, dsts, zeros2, out, idx_s, idx_d, rows, acc, sem):
    cid = lax.axis_index("c")
    sid = lax.axis_index("s")
    wid = cid * 16 + sid
    r0 = sid * RPT
    pltpu.sync_copy(zeros2, acc.at[pl.ds(r0, RPT)])
    plsc.subcore_barrier()

    def body(c, _):
        pltpu.sync_copy(srcs.at[wid, c], idx_s.at[0])
        pltpu.sync_copy(dsts.at[wid, c], idx_d.at[0])
        pltpu.async_copy(y.at[idx_s.at[0]], rows, sem).wait()
        pltpu.sync_copy(rows, acc.at[idx_d.at[0]], add=True)
        return 0

    lax.fori_loop(0, NCH, body, 0)
    plsc.subcore_barrier()
    pltpu.sync_copy(acc.at[pl.ds(r0, RPT)], out.at[cid, pl.ds(r0, RPT)])


_R = 1000


def _rs(da, db):
    return lax.rsqrt(jnp.clip(da + db, 1.0, None))


def _mm_scale(x, w, da, db):
    k = x.shape[1]

    def kern(x_ref, w_ref, da_ref, db_ref, y_ref):
        rs = _rs(da_ref[...], db_ref[...])
        y_ref[...] = jnp.dot(x_ref[...], w_ref[...],
                             preferred_element_type=jnp.float32) * rs

    return pl.pallas_call(
        kern,
        grid=(N // _R,),
        in_specs=[
            pl.BlockSpec((_R, k), lambda i: (i, 0)),
            pl.BlockSpec((k, H), lambda i: (0, 0)),
            pl.BlockSpec((_R, 1), lambda i: (i, 0)),
            pl.BlockSpec((_R, 1), lambda i: (i, 0)),
        ],
        out_specs=pl.BlockSpec((_R, H), lambda i: (i, 0)),
        out_shape=jax.ShapeDtypeStruct((N, H), jnp.float32),
    )(x, w, da, db)


def _fuse(a0, a1, dd_a, dd_b, b1, w2, ds_a, ds_b):
    def kern(a0_ref, a1_ref, dda, ddb, b1_ref, w2_ref, dsa, dsb, y_ref):
        rsd = _rs(dda[...], ddb[...])
        h = jnp.maximum((a0_ref[...] + a1_ref[...]) * rsd + b1_ref[...], 0.0)
        rss = _rs(dsa[...], dsb[...])
        y_ref[...] = jnp.dot(h, w2_ref[...],
                             preferred_element_type=jnp.float32) * rss

    return pl.pallas_call(
        kern,
        grid=(N // _R,),
        in_specs=[
            pl.BlockSpec((_R, H), lambda i: (i, 0)),
            pl.BlockSpec((_R, H), lambda i: (i, 0)),
            pl.BlockSpec((_R, 1), lambda i: (i, 0)),
            pl.BlockSpec((_R, 1), lambda i: (i, 0)),
            pl.BlockSpec((1, H), lambda i: (0, 0)),
            pl.BlockSpec((H, H), lambda i: (0, 0)),
            pl.BlockSpec((_R, 1), lambda i: (i, 0)),
            pl.BlockSpec((_R, 1), lambda i: (i, 0)),
        ],
        out_specs=pl.BlockSpec((_R, H), lambda i: (i, 0)),
        out_shape=jax.ShapeDtypeStruct((N, H), jnp.float32),
    )(a0, a1, dd_a, dd_b, b1, w2, ds_a, ds_b)


def _proj(a0, a1, dd_a, dd_b, b2, p, pb):
    def kern(a0_ref, a1_ref, dda, ddb, b2_ref, p_ref, pb_ref, dx_ref):
        rsd = _rs(dda[...], ddb[...])
        h2 = (a0_ref[...] + a1_ref[...]) * rsd + b2_ref[...]
        dx_ref[...] = jnp.dot(h2, p_ref[...],
                              preferred_element_type=jnp.float32) + pb_ref[...]

    return pl.pallas_call(
        kern,
        grid=(N // _R,),
        in_specs=[
            pl.BlockSpec((_R, H), lambda i: (i, 0)),
            pl.BlockSpec((_R, H), lambda i: (i, 0)),
            pl.BlockSpec((_R, 1), lambda i: (i, 0)),
            pl.BlockSpec((_R, 1), lambda i: (i, 0)),
            pl.BlockSpec((1, H), lambda i: (0, 0)),
            pl.BlockSpec((H, OUT), lambda i: (0, 0)),
            pl.BlockSpec((1, OUT), lambda i: (0, 0)),
        ],
        out_specs=pl.BlockSpec((_R, OUT), lambda i: (i, 0)),
        out_shape=jax.ShapeDtypeStruct((N, OUT), jnp.float32),
    )(a0, a1, dd_a, dd_b, b2, p, pb)


def kernel(t, x_user, x_item, edge_index_u2i, edge_index_i2u,
           W1_u2i, b1_u2i, W1_i2u, b1_i2u,
           W2_u2i, b2_u2i, W2_i2u, b2_i2u,
           P_user, pb_user, P_item, pb_item):
    f32 = jnp.float32

    def pad_idx(row, pad_vals):
        a = row.astype(jnp.int32).reshape(NW, EW)
        p = jnp.broadcast_to(pad_vals[:, None], (NW, EPAD - EW))
        return jnp.concatenate([a, p], axis=1).reshape(NW, NCH, C)

    oob = 25024 + jnp.arange(NW, dtype=jnp.int32)
    inb = jnp.arange(NW, dtype=jnp.int32)
    su_h = pad_idx(edge_index_u2i[0], oob)
    su_s = pad_idx(edge_index_u2i[0], inb)
    du_p = pad_idx(edge_index_u2i[1], oob)
    si_h = pad_idx(edge_index_i2u[0], oob)
    si_s = pad_idx(edge_index_i2u[0], inb)
    di_p = pad_idx(edge_index_i2u[1], oob)
    zeros1 = jnp.zeros((RPT,), f32)
    zeros2 = jnp.zeros((RPT, H), f32)

    hist = _hist(su_h, du_p, si_h, di_p, zeros1)

    def dv(k, a):
        return hist[k, a, :N, None]

    d_su = (dv(0, 0), dv(1, 0))
    d_du = (dv(0, 1), dv(1, 1))
    d_si = (dv(0, 2), dv(1, 2))
    d_di = (dv(0, 3), dv(1, 3))

    y1u = _mm_scale(x_user, W1_u2i, *d_su)
    y1i = _mm_scale(x_item, W1_i2u, *d_si)
    acc_i1 = _scatter(y1u, su_s, du_p, zeros2)
    acc_u1 = _scatter(y1i, si_s, di_p, zeros2)

    y2u = _fuse(acc_u1[0, :N], acc_u1[1, :N], *d_di,
                b1_i2u.reshape(1, H), W2_u2i, *d_su)
    y2i = _fuse(acc_i1[0, :N], acc_i1[1, :N], *d_du,
                b1_u2i.reshape(1, H), W2_i2u, *d_si)
    acc_i2 = _scatter(y2u, su_s, du_p, zeros2)
    acc_u2 = _scatter(y2i, si_s, di_p, zeros2)

    dx_user = _proj(acc_u2[0, :N], acc_u2[1, :N], *d_di,
                    b2_i2u.reshape(1, H), P_user, pb_user.reshape(1, OUT))
    dx_item = _proj(acc_i2[0, :N], acc_i2[1, :N], *d_du,
                    b2_u2i.reshape(1, H), P_item, pb_item.reshape(1, OUT))
    return (dx_user, dx_item)


# speedup vs baseline: 21.6222x; 1.6661x over previous
import functools

import jax
import jax.numpy as jnp
from jax import lax
from jax.experimental import pallas as pl
from jax.experimental.pallas import tpu as pltpu
from jax.experimental.pallas import tpu_sc as plsc

N = 25000
NPAD = 25088
E = 400000
D = 128
H = 64
OUT = 2

NW = 32
EW = E // NW
C = 128
NCH = 98
NCHP = NCH + 3
EPAD = NCH * C
RPT = NPAD // 16

_mesh = plsc.VectorSubcoreMesh(core_axis_name="c", subcore_axis_name="s")
_sc_params = pltpu.CompilerParams(use_tc_tiling_on_sc=False)


@functools.partial(
    pl.kernel,
    out_type=jax.ShapeDtypeStruct((2, 4, NPAD), jnp.float32),
    mesh=_mesh,
    scratch_types=[
        pltpu.VMEM((NCH, C), jnp.int32),
        pltpu.VMEM((128,), jnp.float32),
        pltpu.VMEM_SHARED((NPAD,), jnp.float32),
        pltpu.VMEM_SHARED((NPAD,), jnp.float32),
        pltpu.VMEM_SHARED((NPAD,), jnp.float32),
        pltpu.VMEM_SHARED((NPAD,), jnp.float32),
    ],
    compiler_params=_sc_params,
)
def _hist(i0, i1, i2, i3, zeros1, out, idx, ones, a0, a1, a2, a3):
    cid = lax.axis_index("c")
    sid = lax.axis_index("s")
    wid = cid * 16 + sid
    r0 = sid * RPT
    accs = (a0, a1, a2, a3)
    for acc in accs:
        pltpu.sync_copy(zeros1, acc.at[pl.ds(r0, RPT)])
    for i in range(8):
        ones[pl.ds(i * 16, 16)] = jnp.full((16,), 1.0, jnp.float32)
    plsc.subcore_barrier()
    for src, acc in zip((i0, i1, i2, i3), accs):
        pltpu.sync_copy(src.at[wid], idx)

        def body(c, _, acc=acc):
            pltpu.sync_copy(ones.at[pl.ds(0, C)], acc.at[idx.at[c]], add=True)
            return 0

        lax.fori_loop(0, NCH, body, 0)
    plsc.subcore_barrier()
    for a, acc in enumerate(accs):
        pltpu.sync_copy(acc.at[pl.ds(r0, RPT)], out.at[cid, a, pl.ds(r0, RPT)])


@functools.partial(
    pl.kernel,
    out_type=jax.ShapeDtypeStruct((2, NPAD, H), jnp.float32),
    mesh=_mesh,
    scratch_types=[
        pltpu.VMEM((4, 2, C), jnp.int32),
        pltpu.VMEM((2, C, H), jnp.float32),
        pltpu.VMEM_SHARED((NPAD, H), jnp.float32),
        pltpu.SemaphoreType.DMA((4,)),
        pltpu.SemaphoreType.DMA((2,)),
    ],
    compiler_params=_sc_params,
)
def _scatter(y, sd, zeros2, out, idx, rows, acc, sem_i, sem_g):
    cid = lax.axis_index("c")
    sid = lax.axis_index("s")
    wid = cid * 16 + sid
    r0 = sid * RPT
    pltpu.sync_copy(zeros2, acc.at[pl.ds(r0, RPT)])
    pltpu.sync_copy(sd.at[wid, 0], idx.at[0])
    pltpu.async_copy(sd.at[wid, 1], idx.at[1], sem_i.at[1])
    pltpu.async_copy(sd.at[wid, 2], idx.at[2], sem_i.at[2])
    pltpu.async_copy(y.at[idx.at[0, 0]], rows.at[0], sem_g.at[0])
    plsc.subcore_barrier()

    def body(j, _):
        k0, b0 = j & 3, j & 1
        k1, b1 = (j + 1) & 3, (j + 1) & 1
        k3 = (j + 3) & 3
        pltpu.make_async_copy(sd.at[wid, j + 1], idx.at[k1], sem_i.at[k1]).wait()
        pltpu.async_copy(y.at[idx.at[k1, 0]], rows.at[b1], sem_g.at[b1])
        pltpu.async_copy(sd.at[wid, j + 3], idx.at[k3], sem_i.at[k3])
        pltpu.make_async_copy(y.at[idx.at[k0, 0]], rows.at[b0], sem_g.at[b0]).wait()
        pltpu.sync_copy(rows.at[b0], acc.at[idx.at[k0, 1]], add=True)
        return 0

    lax.fori_loop(0, NCH, body, 0)
    pltpu.make_async_copy(y.at[idx.at[NCH & 3, 0]], rows.at[NCH & 1],
                          sem_g.at[NCH & 1]).wait()
    for c in (NCH + 1, NCH + 2):
        pltpu.make_async_copy(sd.at[wid, c], idx.at[c & 3], sem_i.at[c & 3]).wait()
    plsc.subcore_barrier()
    pltpu.sync_copy(acc.at[pl.ds(r0, RPT)], out.at[cid, pl.ds(r0, RPT)])


_R = 1000


def _rs(da, db):
    return lax.rsqrt(jnp.clip(da + db, 1.0, None))


def _mm_scale(x, w, da, db):
    k = x.shape[1]

    def kern(x_ref, w_ref, da_ref, db_ref, y_ref):
        rs = _rs(da_ref[...], db_ref[...])
        y_ref[...] = jnp.dot(x_ref[...], w_ref[...],
                             preferred_element_type=jnp.float32) * rs

    return pl.pallas_call(
        kern,
        grid=(N // _R,),
        in_specs=[
            pl.BlockSpec((_R, k), lambda i: (i, 0)),
            pl.BlockSpec((k, H), lambda i: (0, 0)),
            pl.BlockSpec((_R, 1), lambda i: (i, 0)),
            pl.BlockSpec((_R, 1), lambda i: (i, 0)),
        ],
        out_specs=pl.BlockSpec((_R, H), lambda i: (i, 0)),
        out_shape=jax.ShapeDtypeStruct((N, H), jnp.float32),
    )(x, w, da, db)


def _fuse(a0, a1, dd_a, dd_b, b1, w2, ds_a, ds_b):
    def kern(a0_ref, a1_ref, dda, ddb, b1_ref, w2_ref, dsa, dsb, y_ref):
        rsd = _rs(dda[...], ddb[...])
        h = jnp.maximum((a0_ref[...] + a1_ref[...]) * rsd + b1_ref[...], 0.0)
        rss = _rs(dsa[...], dsb[...])
        y_ref[...] = jnp.dot(h, w2_ref[...],
                             preferred_element_type=jnp.float32) * rss

    return pl.pallas_call(
        kern,
        grid=(N // _R,),
        in_specs=[
            pl.BlockSpec((_R, H), lambda i: (i, 0)),
            pl.BlockSpec((_R, H), lambda i: (i, 0)),
            pl.BlockSpec((_R, 1), lambda i: (i, 0)),
            pl.BlockSpec((_R, 1), lambda i: (i, 0)),
            pl.BlockSpec((1, H), lambda i: (0, 0)),
            pl.BlockSpec((H, H), lambda i: (0, 0)),
            pl.BlockSpec((_R, 1), lambda i: (i, 0)),
            pl.BlockSpec((_R, 1), lambda i: (i, 0)),
        ],
        out_specs=pl.BlockSpec((_R, H), lambda i: (i, 0)),
        out_shape=jax.ShapeDtypeStruct((N, H), jnp.float32),
    )(a0, a1, dd_a, dd_b, b1, w2, ds_a, ds_b)


def _proj(a0, a1, dd_a, dd_b, b2, p, pb):
    def kern(a0_ref, a1_ref, dda, ddb, b2_ref, p_ref, pb_ref, dx_ref):
        rsd = _rs(dda[...], ddb[...])
        h2 = (a0_ref[...] + a1_ref[...]) * rsd + b2_ref[...]
        dx_ref[...] = jnp.dot(h2, p_ref[...],
                              preferred_element_type=jnp.float32) + pb_ref[...]

    return pl.pallas_call(
        kern,
        grid=(N // _R,),
        in_specs=[
            pl.BlockSpec((_R, H), lambda i: (i, 0)),
            pl.BlockSpec((_R, H), lambda i: (i, 0)),
            pl.BlockSpec((_R, 1), lambda i: (i, 0)),
            pl.BlockSpec((_R, 1), lambda i: (i, 0)),
            pl.BlockSpec((1, H), lambda i: (0, 0)),
            pl.BlockSpec((H, OUT), lambda i: (0, 0)),
            pl.BlockSpec((1, OUT), lambda i: (0, 0)),
        ],
        out_specs=pl.BlockSpec((_R, OUT), lambda i: (i, 0)),
        out_shape=jax.ShapeDtypeStruct((N, OUT), jnp.float32),
    )(a0, a1, dd_a, dd_b, b2, p, pb)


def kernel(t, x_user, x_item, edge_index_u2i, edge_index_i2u,
           W1_u2i, b1_u2i, W1_i2u, b1_i2u,
           W2_u2i, b2_u2i, W2_i2u, b2_i2u,
           P_user, pb_user, P_item, pb_item):
    f32 = jnp.float32

    oob = 25024 + jnp.arange(NW, dtype=jnp.int32)
    inb = jnp.arange(NW, dtype=jnp.int32)

    def pad_idx(row, pad_vals, nch):
        a = row.astype(jnp.int32).reshape(NW, EW)
        p = jnp.broadcast_to(pad_vals[:, None], (NW, nch * C - EW))
        return jnp.concatenate([a, p], axis=1).reshape(NW, nch, C)

    def pack_sd(e):
        s = pad_idx(e[0], inb, NCHP)
        d = pad_idx(e[1], oob, NCHP)
        return jnp.stack([s, d], axis=2)

    sd_u2i = pack_sd(edge_index_u2i)
    sd_i2u = pack_sd(edge_index_i2u)
    su_h = pad_idx(edge_index_u2i[0], oob, NCH)
    du_h = pad_idx(edge_index_u2i[1], oob, NCH)
    si_h = pad_idx(edge_index_i2u[0], oob, NCH)
    di_h = pad_idx(edge_index_i2u[1], oob, NCH)
    zeros1 = jnp.zeros((RPT,), f32)
    zeros2 = jnp.zeros((RPT, H), f32)

    hist = _hist(su_h, du_h, si_h, di_h, zeros1)

    def dv(k, a):
        return hist[k, a, :N, None]

    d_su = (dv(0, 0), dv(1, 0))
    d_du = (dv(0, 1), dv(1, 1))
    d_si = (dv(0, 2), dv(1, 2))
    d_di = (dv(0, 3), dv(1, 3))

    y1u = _mm_scale(x_user, W1_u2i, *d_su)
    y1i = _mm_scale(x_item, W1_i2u, *d_si)
    acc_i1 = _scatter(y1u, sd_u2i, zeros2)
    acc_u1 = _scatter(y1i, sd_i2u, zeros2)

    y2u = _fuse(acc_u1[0, :N], acc_u1[1, :N], *d_di,
                b1_i2u.reshape(1, H), W2_u2i, *d_su)
    y2i = _fuse(acc_i1[0, :N], acc_i1[1, :N], *d_du,
                b1_u2i.reshape(1, H), W2_i2u, *d_si)
    acc_i2 = _scatter(y2u, sd_u2i, zeros2)
    acc_u2 = _scatter(y2i, sd_i2u, zeros2)

    dx_user = _proj(acc_u2[0, :N], acc_u2[1, :N], *d_di,
                    b2_i2u.reshape(1, H), P_user, pb_user.reshape(1, OUT))
    dx_item = _proj(acc_i2[0, :N], acc_i2[1, :N], *d_du,
                    b2_u2i.reshape(1, H), P_item, pb_item.reshape(1, OUT))
    return (dx_user, dx_item)
